# baseline (speedup 1.0000x reference)
import jax
import jax.numpy as jnp
from jax.experimental import pallas as pl
from jax.experimental.pallas import tpu as pltpu

_D = 1024
_E = 16
_F = 4096
_FT = 2048
_NF = _F // _FT
_INV_SQRT2 = 0.7071067811865476


def _moe_step(x_ref, wr_ref, w1_ref, b1_ref, w2_ref, b2_ref, out_ref, acc_ref):
    e = pl.program_id(0)
    f = pl.program_id(1)
    x = x_ref[...]

    h = jax.lax.dot_general(x, w1_ref[0], (((1,), (1,)), ((), ())),
                            preferred_element_type=jnp.float32)
    h = h + b1_ref[0]
    a = 0.5 * h * (1.0 + jax.lax.erf(h * _INV_SQRT2))
    w2s = w2_ref[0, :, pl.ds(f * _FT, _FT)]
    partial = jax.lax.dot_general(a, w2s, (((1,), (1,)), ((), ())),
                                  preferred_element_type=jnp.float32)

    @pl.when(f == 0)
    def _init_acc():
        acc_ref[...] = partial

    @pl.when(f > 0)
    def _acc():
        acc_ref[...] += partial

    @pl.when(f == _NF - 1)
    def _combine():
        logits = jax.lax.dot_general(x, wr_ref[...], (((1,), (1,)), ((), ())),
                                     preferred_element_type=jnp.float32)
        col = jax.lax.broadcasted_iota(jnp.int32, logits.shape, 1)
        m1 = jnp.max(logits, axis=-1)
        a1 = jnp.min(jnp.where(logits == m1[:, None], col, _E), axis=-1)
        neg = jnp.finfo(jnp.float32).min
        logits2 = jnp.where(col == a1[:, None], neg, logits)
        m2 = jnp.max(logits2, axis=-1)
        a2 = jnp.min(jnp.where(logits2 == m2[:, None], col, _E), axis=-1)
        p1 = 1.0 / (1.0 + jnp.exp(m2 - m1))
        w_e = jnp.where(a1 == e, p1, 0.0) + jnp.where(a2 == e, 1.0 - p1, 0.0)

        contrib = w_e[:, None] * (acc_ref[...] + b2_ref[0])

        @pl.when(e == 0)
        def _init_out():
            out_ref[...] = contrib

        @pl.when(e > 0)
        def _acc_out():
            out_ref[...] += contrib


def kernel(x, Wr, W1, b1, W2, b2):
    B, S, D = x.shape
    N = B * S
    xf = x.reshape(N, D)
    b1r = b1.reshape(_E, 1, _F)
    b2r = b2.reshape(_E, 1, _D)
    out = pl.pallas_call(
        _moe_step,
        grid=(_E, _NF),
        in_specs=[
            pl.BlockSpec((N, D), lambda e, f: (0, 0)),
            pl.BlockSpec((_E, D), lambda e, f: (0, 0)),
            pl.BlockSpec((1, _FT, _D), lambda e, f: (e, f, 0)),
            pl.BlockSpec((1, 1, _FT), lambda e, f: (e, 0, f)),
            pl.BlockSpec((1, _D, _F), lambda e, f: (e, 0, 0)),
            pl.BlockSpec((1, 1, _D), lambda e, f: (e, 0, 0)),
        ],
        out_specs=pl.BlockSpec((N, D), lambda e, f: (0, 0)),
        out_shape=jax.ShapeDtypeStruct((N, D), jnp.float32),
        scratch_shapes=[pltpu.VMEM((N, _D), jnp.float32)],
        compiler_params=pltpu.CompilerParams(
            dimension_semantics=("arbitrary", "arbitrary"),
        ),
    )(xf, Wr, W1, b1r, W2, b2r)
    return out.reshape(B, S, D)


# speedup vs baseline: 1.0326x; 1.0326x over previous
import functools

import jax
import jax.numpy as jnp
from jax import lax
from jax.experimental import pallas as pl
from jax.experimental.pallas import tpu as pltpu
from jax.experimental.pallas import tpu_sc as plsc

_D = 1024
_E = 16
_F = 4096
_FT = 2048
_NF = _F // _FT
_N = 256
_INV_SQRT2 = 0.7071067811865476
_NC = 2
_NS = 16
_TPW = _N // (_NC * _NS)


def _logits_step(x_ref, wr_ref, out_ref):
    out_ref[...] = jax.lax.dot_general(
        x_ref[...], wr_ref[...], (((1,), (1,)), ((), ())),
        preferred_element_type=jnp.float32)


def _router_logits(xf, Wr):
    return pl.pallas_call(
        _logits_step,
        out_shape=jax.ShapeDtypeStruct((_N, _E), jnp.float32),
    )(xf, Wr)


def _rot(v, k):
    idx = jnp.remainder(lax.iota(jnp.int32, _E) + k, _E)
    dnums = lax.GatherDimensionNumbers(
        offset_dims=(), collapsed_slice_dims=(0,), start_index_map=(0,))
    return lax.gather(v, idx[:, None], dnums, slice_sizes=(1,),
                      mode=lax.GatherScatterMode.PROMISE_IN_BOUNDS)


def _allmax(v):
    for k in (8, 4, 2, 1):
        v = jnp.maximum(v, _rot(v, k))
    return v


def _allmin(v):
    for k in (8, 4, 2, 1):
        v = jnp.minimum(v, _rot(v, k))
    return v


def _route_body(lg_hbm, w_hbm, lg_v, w_v):
    wid = lax.axis_index("s") * _NC + lax.axis_index("c")
    base = wid * _TPW
    pltpu.sync_copy(lg_hbm.at[pl.ds(base, _TPW)], lg_v)
    col = lax.iota(jnp.int32, _E)
    for t in range(_TPW):
        lg = lg_v[t]
        m1 = _allmax(lg)
        a1 = _allmin(jnp.where(lg == m1, col, _E))
        lg2 = jnp.where(col == a1, jnp.float32(-3.4e38), lg)
        m2 = _allmax(lg2)
        a2 = _allmin(jnp.where(lg2 == m2, col, _E))
        p1 = 1.0 / (1.0 + jnp.exp(m2 - m1))
        w_v[t] = (jnp.where(col == a1, p1, 0.0)
                  + jnp.where(col == a2, 1.0 - p1, 0.0))
    pltpu.sync_copy(w_v, w_hbm.at[pl.ds(base, _TPW)])


def _route_sc(logits):
    mesh = plsc.VectorSubcoreMesh(core_axis_name="c", subcore_axis_name="s",
                                  num_cores=_NC, num_subcores=_NS)
    fn = pl.kernel(
        _route_body,
        mesh=mesh,
        out_type=jax.ShapeDtypeStruct((_N, _E), jnp.float32),
        scratch_types=[
            pltpu.VMEM((_TPW, _E), jnp.float32),
            pltpu.VMEM((_TPW, _E), jnp.float32),
        ],
    )
    return fn(logits)


def _moe_step(x_ref, w_ref, w1_ref, b1_ref, w2_ref, b2_ref, out_ref, acc_ref):
    e = pl.program_id(0)
    f = pl.program_id(1)
    x = x_ref[...]

    h = jax.lax.dot_general(x, w1_ref[0], (((1,), (1,)), ((), ())),
                            preferred_element_type=jnp.float32)
    h = h + b1_ref[0]
    a = 0.5 * h * (1.0 + jax.lax.erf(h * _INV_SQRT2))
    partial = jax.lax.dot_general(a, w2_ref[0], (((1,), (1,)), ((), ())),
                                  preferred_element_type=jnp.float32)

    @pl.when(f == 0)
    def _init_acc():
        acc_ref[...] = partial

    @pl.when(f > 0)
    def _acc():
        acc_ref[...] += partial

    @pl.when(f == _NF - 1)
    def _combine():
        col = jax.lax.broadcasted_iota(jnp.int32, (_N, _E), 1)
        w_e = jnp.sum(jnp.where(col == e, w_ref[...], 0.0), axis=-1)
        contrib = w_e[:, None] * (acc_ref[...] + b2_ref[0])

        @pl.when(e == 0)
        def _init_out():
            out_ref[...] = contrib

        @pl.when(e > 0)
        def _acc_out():
            out_ref[...] += contrib


def kernel(x, Wr, W1, b1, W2, b2):
    B, S, D = x.shape
    xf = x.reshape(_N, D)
    b1r = b1.reshape(_E, 1, _F)
    b2r = b2.reshape(_E, 1, _D)

    w_all = _route_sc(_router_logits(xf, Wr))

    out = pl.pallas_call(
        _moe_step,
        grid=(_E, _NF),
        in_specs=[
            pl.BlockSpec((_N, D), lambda e, f: (0, 0)),
            pl.BlockSpec((_N, _E), lambda e, f: (0, 0)),
            pl.BlockSpec((1, _FT, _D), lambda e, f: (e, f, 0)),
            pl.BlockSpec((1, 1, _FT), lambda e, f: (e, 0, f)),
            pl.BlockSpec((1, _D, _FT), lambda e, f: (e, 0, f)),
            pl.BlockSpec((1, 1, _D), lambda e, f: (e, 0, 0)),
        ],
        out_specs=pl.BlockSpec((_N, _D), lambda e, f: (0, 0)),
        out_shape=jax.ShapeDtypeStruct((_N, _D), jnp.float32),
        scratch_shapes=[pltpu.VMEM((_N, _D), jnp.float32)],
        compiler_params=pltpu.CompilerParams(
            dimension_semantics=("arbitrary", "arbitrary"),
        ),
    )(xf, w_all, W1, b1r, W2, b2r)
    return out.reshape(B, S, D)


# speedup vs baseline: 1.1208x; 1.0855x over previous
import jax
import jax.numpy as jnp
from jax.experimental import pallas as pl
from jax.experimental.pallas import tpu as pltpu

_D = 1024
_E = 16
_F = 4096
_FT = 2048
_NF = _F // _FT
_INV_SQRT2 = 0.7071067811865476


def _moe_step(x_ref, wr_ref, w1_ref, b1_ref, w2_ref, b2_ref, out_ref, acc_ref,
              w_scr):
    e = pl.program_id(0)
    f = pl.program_id(1)
    x = x_ref[...]

    @pl.when((e == 0) & (f == 0))
    def _routing():
        logits = jax.lax.dot_general(x, wr_ref[...], (((1,), (1,)), ((), ())),
                                     preferred_element_type=jnp.float32)
        col = jax.lax.broadcasted_iota(jnp.int32, logits.shape, 1)
        m1 = jnp.max(logits, axis=-1)
        a1 = jnp.min(jnp.where(logits == m1[:, None], col, _E), axis=-1)
        neg = jnp.finfo(jnp.float32).min
        logits2 = jnp.where(col == a1[:, None], neg, logits)
        m2 = jnp.max(logits2, axis=-1)
        a2 = jnp.min(jnp.where(logits2 == m2[:, None], col, _E), axis=-1)
        p1 = 1.0 / (1.0 + jnp.exp(m2 - m1))
        w_scr[...] = (jnp.where(col == a1[:, None], p1[:, None], 0.0)
                      + jnp.where(col == a2[:, None], (1.0 - p1)[:, None], 0.0))

    h = jax.lax.dot_general(x, w1_ref[0], (((1,), (1,)), ((), ())),
                            preferred_element_type=jnp.float32)
    h = h + b1_ref[0]
    a = 0.5 * h * (1.0 + jax.lax.erf(h * _INV_SQRT2))
    partial = jax.lax.dot_general(a, w2_ref[0], (((1,), (1,)), ((), ())),
                                  preferred_element_type=jnp.float32)

    @pl.when(f == 0)
    def _init_acc():
        acc_ref[...] = partial

    @pl.when(f > 0)
    def _acc():
        acc_ref[...] += partial

    @pl.when(f == _NF - 1)
    def _combine():
        col = jax.lax.broadcasted_iota(jnp.int32, (256, _E), 1)
        w_e = jnp.sum(jnp.where(col == e, w_scr[...], 0.0), axis=-1)
        contrib = w_e[:, None] * (acc_ref[...] + b2_ref[0])

        @pl.when(e == 0)
        def _init_out():
            out_ref[...] = contrib

        @pl.when(e > 0)
        def _acc_out():
            out_ref[...] += contrib


def kernel(x, Wr, W1, b1, W2, b2):
    B, S, D = x.shape
    N = B * S
    xf = x.reshape(N, D)
    b1r = b1.reshape(_E, 1, _F)
    b2r = b2.reshape(_E, 1, _D)
    out = pl.pallas_call(
        _moe_step,
        grid=(_E, _NF),
        in_specs=[
            pl.BlockSpec((N, D), lambda e, f: (0, 0)),
            pl.BlockSpec((_E, D), lambda e, f: (0, 0)),
            pl.BlockSpec((1, _FT, _D), lambda e, f: (e, f, 0)),
            pl.BlockSpec((1, 1, _FT), lambda e, f: (e, 0, f)),
            pl.BlockSpec((1, _D, _FT), lambda e, f: (e, 0, f)),
            pl.BlockSpec((1, 1, _D), lambda e, f: (e, 0, 0)),
        ],
        out_specs=pl.BlockSpec((N, D), lambda e, f: (0, 0)),
        out_shape=jax.ShapeDtypeStruct((N, D), jnp.float32),
        scratch_shapes=[pltpu.VMEM((N, _D), jnp.float32),
                        pltpu.VMEM((N, _E), jnp.float32)],
        compiler_params=pltpu.CompilerParams(
            dimension_semantics=("arbitrary", "arbitrary"),
        ),
    )(xf, Wr, W1, b1r, W2, b2r)
    return out.reshape(B, S, D)
